# baseline (speedup 1.0000x reference)
import functools

import jax
import jax.numpy as jnp
from jax import lax
from jax.experimental import pallas as pl
from jax.experimental.pallas import tpu as pltpu
from jax.experimental.pallas import tpu_sc as plsc


NW = 32
NPIX = 4 * 512 * 512
PX_PER_W = NPIX // NW
CHUNK = 2048
NCHUNK = PX_PER_W // CHUNK
NIT = CHUNK // 16
GRID_W = 24576
C = 12

_COFF = (0, 1, 8, 9, 128, 129, 136, 137)


def _sc_body(a_hbm, g_hbm, out_hbm, a_v, g_v, o_v):
    wid = lax.axis_index("s") * 2 + lax.axis_index("c")
    b = wid // 8
    t = wid % 8
    pltpu.sync_copy(a_hbm.at[pl.ds(b * GRID_W, GRID_W)], a_v)
    lane = lax.iota(jnp.int32, 16)

    def chunk_body(ci, carry):
        p_in_b = t * PX_PER_W + ci * CHUNK
        g_off = b * (NPIX // 4) + p_in_b
        pltpu.sync_copy(g_hbm.at[pl.ds(pl.multiple_of(g_off, CHUNK), CHUNK)], g_v)

        def it_body(ii, carry2):
            pvec = (p_in_b + ii * 16) + lane
            hv = lax.shift_right_logical(pvec, 9)
            wv = jnp.bitwise_and(pvec, 511)
            tx = hv.astype(jnp.float32) * jnp.float32(15.0 / 511.0)
            i0 = jnp.minimum(tx.astype(jnp.int32), 14)
            fx = tx - i0.astype(jnp.float32)
            ty = wv.astype(jnp.float32) * jnp.float32(15.0 / 511.0)
            j0 = jnp.minimum(ty.astype(jnp.int32), 14)
            fy = ty - j0.astype(jnp.float32)
            g = g_v[pl.ds(pl.multiple_of(ii * 16, 16), 16)]
            tz = jnp.clip((g + 1.0) * 3.5, 0.0, 7.0)
            k0 = jnp.minimum(tz.astype(jnp.int32), 6)
            fz = tz - k0.astype(jnp.float32)

            base = (i0 * 16 + j0) * 8 + k0
            wx0, wy0, wz0 = 1.0 - fx, 1.0 - fy, 1.0 - fz
            w00, w01 = wx0 * wy0, wx0 * fy
            w10, w11 = fx * wy0, fx * fy
            cw = (w00 * wz0, w00 * fz, w01 * wz0, w01 * fz,
                  w10 * wz0, w10 * fz, w11 * wz0, w11 * fz)

            oidx = (ii * 16) * C + lane * C
            for c in range(C):
                idx0 = base + c * 2048
                acc = cw[0] * plsc.load_gather(a_v, [idx0])
                for k in range(1, 8):
                    acc = acc + cw[k] * plsc.load_gather(a_v, [idx0 + _COFF[k]])
                plsc.store_scatter(o_v, [oidx + c], acc)
            return carry2

        lax.fori_loop(0, NIT, it_body, 0)
        pltpu.sync_copy(
            o_v, out_hbm.at[pl.ds(pl.multiple_of(g_off * C, CHUNK), CHUNK * C)])
        return carry

    lax.fori_loop(0, NCHUNK, chunk_body, 0)


@jax.jit
def kernel(A, guide):
    bs, H, W, _ = guide.shape
    a_flat = A.reshape(bs * GRID_W)
    g_flat = guide.reshape(bs * H * W)

    mesh = plsc.VectorSubcoreMesh(
        core_axis_name="c", subcore_axis_name="s", num_cores=2, num_subcores=16)
    sc_slice = functools.partial(
        pl.kernel,
        out_type=jax.ShapeDtypeStruct((bs * H * W * C,), jnp.float32),
        mesh=mesh,
        compiler_params=pltpu.CompilerParams(
            use_tc_tiling_on_sc=False, needs_layout_passes=False),
        scratch_types=[
            pltpu.VMEM((GRID_W,), jnp.float32),
            pltpu.VMEM((CHUNK,), jnp.float32),
            pltpu.VMEM((CHUNK * C,), jnp.float32),
        ],
    )(_sc_body)

    out = sc_slice(a_flat, g_flat)
    return out.reshape(bs, H, W, C)


# speedup vs baseline: 2.4875x; 2.4875x over previous
import functools

import jax
import jax.numpy as jnp
from jax import lax
from jax.experimental import pallas as pl
from jax.experimental.pallas import tpu as pltpu
from jax.experimental.pallas import tpu_sc as plsc


NW = 32
NPIX = 4 * 512 * 512
PPB = NPIX // 4
PX_PER_W = NPIX // NW
CHUNK = 2048
NCHUNK = PX_PER_W // CHUNK
NIT = CHUNK // 16
GRID_W = 24576
C = 12

_COFF = (0, 1, 8, 9, 128, 129, 136, 137)


def _sc_body(a_hbm, g_hbm, out_hbm, a_v, g_v, o_v, sem):
    wid = lax.axis_index("s") * 2 + lax.axis_index("c")
    b = wid // 8
    t = wid % 8
    pltpu.sync_copy(a_hbm.at[pl.ds(b * GRID_W, GRID_W)], a_v)
    lane = lax.iota(jnp.int32, 16)

    def chunk_body(ci, carry):
        p_in_b = t * PX_PER_W + ci * CHUNK
        pltpu.sync_copy(
            g_hbm.at[pl.ds(pl.multiple_of(b * PPB + p_in_b, CHUNK), CHUNK)], g_v)

        def it_body(ii, carry2):
            pvec = (p_in_b + ii * 16) + lane
            hv = lax.shift_right_logical(pvec, 9)
            wv = jnp.bitwise_and(pvec, 511)
            tx = hv.astype(jnp.float32) * jnp.float32(15.0 / 511.0)
            i0 = jnp.minimum(tx.astype(jnp.int32), 14)
            fx = tx - i0.astype(jnp.float32)
            ty = wv.astype(jnp.float32) * jnp.float32(15.0 / 511.0)
            j0 = jnp.minimum(ty.astype(jnp.int32), 14)
            fy = ty - j0.astype(jnp.float32)
            g = g_v[pl.ds(pl.multiple_of(ii * 16, 16), 16)]
            tz = jnp.clip((g + 1.0) * 3.5, 0.0, 7.0)
            k0 = jnp.minimum(tz.astype(jnp.int32), 6)
            fz = tz - k0.astype(jnp.float32)

            base = (i0 * 16 + j0) * 8 + k0
            wx0, wy0, wz0 = 1.0 - fx, 1.0 - fy, 1.0 - fz
            w00, w01 = wx0 * wy0, wx0 * fy
            w10, w11 = fx * wy0, fx * fy
            cw = (w00 * wz0, w00 * fz, w01 * wz0, w01 * fz,
                  w10 * wz0, w10 * fz, w11 * wz0, w11 * fz)

            for c in range(C):
                idx0 = base + c * 2048
                v = [plsc.load_gather(a_v, [idx0 + _COFF[k]]) for k in range(8)]
                t0 = cw[0] * v[0] + cw[1] * v[1]
                t1 = cw[2] * v[2] + cw[3] * v[3]
                t2 = cw[4] * v[4] + cw[5] * v[5]
                t3 = cw[6] * v[6] + cw[7] * v[7]
                o_v[c, pl.ds(pl.multiple_of(ii * 16, 16), 16)] = (
                    (t0 + t1) + (t2 + t3))
            return carry2

        lax.fori_loop(0, NIT, it_body, 0)
        copies = [
            pltpu.async_copy(
                o_v.at[c],
                out_hbm.at[pl.ds(
                    pl.multiple_of((b * C + c) * PPB + p_in_b, CHUNK), CHUNK)],
                sem)
            for c in range(C)
        ]
        for cp in copies:
            cp.wait()
        return carry

    lax.fori_loop(0, NCHUNK, chunk_body, 0)


@jax.jit
def kernel(A, guide):
    bs, H, W, _ = guide.shape
    a_flat = A.reshape(bs * GRID_W)
    g_flat = guide.reshape(bs * H * W)

    mesh = plsc.VectorSubcoreMesh(
        core_axis_name="c", subcore_axis_name="s", num_cores=2, num_subcores=16)
    sc_slice = functools.partial(
        pl.kernel,
        out_type=jax.ShapeDtypeStruct((bs * C * H * W,), jnp.float32),
        mesh=mesh,
        compiler_params=pltpu.CompilerParams(needs_layout_passes=False),
        scratch_types=[
            pltpu.VMEM((GRID_W,), jnp.float32),
            pltpu.VMEM((CHUNK,), jnp.float32),
            pltpu.VMEM((C, CHUNK), jnp.float32),
            pltpu.SemaphoreType.DMA,
        ],
    )(_sc_body)

    out = sc_slice(a_flat, g_flat)
    return jnp.transpose(out.reshape(bs, C, H, W), (0, 2, 3, 1))
